# baseline (speedup 1.0000x reference)
import jax
import jax.numpy as jnp
from jax import lax
from jax.experimental import pallas as pl
from jax.experimental.pallas import tpu as pltpu


def _ceil_to(x, m):
    return ((x + m - 1) // m) * m


def _pool_kernel(wi_ref, mwl_ref, hs_ref, emb_ref, masks_ref):
    wi = wi_ref[...]
    starts = wi[:, :, 0:1]
    spans = wi[:, :, 1:2] - starts
    Bb, Wp, _ = wi.shape

    mwl = mwl_ref[0]

    wio_l = lax.broadcasted_iota(jnp.int32, (Bb, 1, Wp), 2)
    valid_l = wio_l < mwl[:, :, None]
    masks_ref[...] = valid_l.astype(jnp.int32)

    wio_s = lax.broadcasted_iota(jnp.int32, (Bb, Wp, 1), 1)
    spans = jnp.where(jnp.logical_and(wio_s < mwl[:, :, None], spans >= 0),
                      spans, -1)

    hs = hs_ref[...]
    Sp = hs.shape[1]

    pos = lax.broadcasted_iota(jnp.int32, (Bb, Wp, Sp), 2)
    rel = pos - starts
    in_span = jnp.logical_and(rel >= 0, rel <= spans)
    sel = jnp.where(in_span, jnp.float32(1.0), jnp.float32(0.0)).astype(jnp.bfloat16)

    pooled = lax.dot_general(
        sel, hs.astype(jnp.bfloat16),
        (((2,), (1,)), ((0,), (0,))),
        preferred_element_type=jnp.float32)

    denom = jnp.maximum(spans + 1, 1).astype(jnp.float32)
    inv = jnp.where(spans >= 0, 1.0 / denom, 0.0)
    emb_ref[...] = pooled * inv


def kernel(hidden_states, word_idxs, max_word_len):
    B, S, H = hidden_states.shape
    W = word_idxs.shape[1]
    out_dtype = hidden_states.dtype

    Hp = _ceil_to(H, 128)
    Wp = _ceil_to(W, 8)
    Sp = _ceil_to(S, 8)

    hs = hidden_states
    if (Sp, Hp) != (S, H):
        hs = jnp.pad(hs, ((0, 0), (0, Sp - S), (0, Hp - H)))

    wi = word_idxs.astype(jnp.int32)
    if Wp != W:
        wi = jnp.pad(wi, ((0, 0), (0, Wp - W), (0, 0)))

    Bb = 4
    while B % Bb != 0:
        Bb //= 2

    mwl = max_word_len.astype(jnp.int32).reshape(B // Bb, Bb, 1)

    emb, masks3 = pl.pallas_call(
        _pool_kernel,
        out_shape=(
            jax.ShapeDtypeStruct((B, Wp, Hp), out_dtype),
            jax.ShapeDtypeStruct((B, 1, Wp), jnp.int32),
        ),
        grid=(B // Bb,),
        in_specs=[
            pl.BlockSpec((Bb, Wp, 2), lambda b: (b, 0, 0)),
            pl.BlockSpec((1, Bb, 1), lambda b: (b, 0, 0)),
            pl.BlockSpec((Bb, Sp, Hp), lambda b: (b, 0, 0)),
        ],
        out_specs=(
            pl.BlockSpec((Bb, Wp, Hp), lambda b: (b, 0, 0)),
            pl.BlockSpec((Bb, 1, Wp), lambda b: (b, 0, 0)),
        ),
        compiler_params=pltpu.CompilerParams(
            dimension_semantics=("parallel",),
            vmem_limit_bytes=56 * 1024 * 1024),
    )(wi, mwl, hs)

    word_masks = masks3[:, 0, :W]
    if (Wp, Hp) != (W, H):
        emb = emb[:, :W, :H]
    return emb, word_masks
